# baseline (speedup 1.0000x reference)
import functools

import jax
import jax.numpy as jnp
from jax import lax
from jax.experimental import pallas as pl
from jax.experimental.pallas import tpu as pltpu
from jax.experimental.pallas import tpu_sc as plsc

_F = 20
_OUT_F = 49
_DIMS = (1, 6, 12, 10)
_FLAT_OFF = (0, 2, 80, 452)
_FLAT_LEN = 692
_NB = 2


def _make_sc_call(b, l):
    info = plsc.get_sparse_core_info()
    nc, ns = info.num_cores, info.num_subcores
    nw = nc * ns
    b_per_w = b // nw
    n_chunks = b_per_w // _NB
    bc = _NB * l
    n = b * l

    mesh = plsc.VectorSubcoreMesh(core_axis_name="c", subcore_axis_name="s")

    @functools.partial(
        pl.kernel,
        mesh=mesh,
        out_type=jax.ShapeDtypeStruct((n * _OUT_F,), jnp.float32),
        compiler_params=pltpu.CompilerParams(
            needs_layout_passes=False, use_tc_tiling_on_sc=True),
        scratch_types=[
            pltpu.VMEM((_FLAT_LEN,), jnp.float32),
            pltpu.VMEM((_NB, l, _F), jnp.float32),
            pltpu.VMEM((bc * _OUT_F,), jnp.float32),
            pltpu.VMEM((4, bc), jnp.int32),
        ],
    )
    def sc_call(x_hbm, t_hbm, out_hbm, t_v, in_v, out_v, base_v):
        wid = lax.axis_index("s") * nc + lax.axis_index("c")
        bbase0 = wid * b_per_w
        pltpu.sync_copy(t_hbm, t_v)

        def chunk_body(ci, carry):
            bbase = bbase0 + ci * _NB
            pltpu.sync_copy(x_hbm.at[pl.ds(bbase, _NB)], in_v)

            @plsc.parallel_loop(0, bc // 16, 1, unroll=4)
            def idx_body(g):
                r16 = lax.iota(jnp.int32, 16) + g * 16
                bv = jnp.where(r16 >= l, 1, 0)
                lv = r16 - bv * l
                for k in range(4):
                    col = jnp.full((16,), 16 + k, jnp.int32)
                    v = plsc.load_gather(in_v, [bv, lv, col]).astype(jnp.int32)
                    base_v[k, pl.ds(g * 16, 16)] = _FLAT_OFF[k] + v * _DIMS[k]

            @plsc.parallel_loop(0, bc // 16, 1, unroll=2)
            def emb_body(g):
                r16 = lax.iota(jnp.int32, 16) + g * 16
                bv = jnp.where(r16 >= l, 1, 0)
                lv = r16 - bv * l
                off49 = r16 * _OUT_F
                for c in range(_F):
                    colv = jnp.full((16,), c, jnp.int32)
                    vals = plsc.load_gather(in_v, [bv, lv, colv])
                    plsc.store_scatter(out_v, [off49 + c], vals)
                c = _F
                for k in range(4):
                    bk = base_v[k, pl.ds(g * 16, 16)]
                    for j in range(_DIMS[k]):
                        vals = plsc.load_gather(t_v, [bk + j])
                        plsc.store_scatter(out_v, [off49 + c], vals)
                        c += 1

            pltpu.sync_copy(
                out_v, out_hbm.at[pl.ds(bbase * l * _OUT_F, bc * _OUT_F)])
            return carry

        lax.fori_loop(0, n_chunks, chunk_body, 0, unroll=False)

    return sc_call


def kernel(data, year_table, month_table, day_table, hour_table):
    b, l, f = data.shape
    t = jnp.concatenate([
        year_table.reshape(-1), month_table.reshape(-1),
        day_table.reshape(-1), hour_table.reshape(-1)])
    out = _make_sc_call(b, l)(data, t)
    return out.reshape(b, l, _OUT_F)


# speedup vs baseline: 1.0164x; 1.0164x over previous
import functools

import jax
import jax.numpy as jnp
from jax import lax
from jax.experimental import pallas as pl
from jax.experimental.pallas import tpu as pltpu
from jax.experimental.pallas import tpu_sc as plsc

_F = 20
_OUT_F = 49
_DIMS = (1, 6, 12, 10)
_FLAT_OFF = (0, 2, 80, 452)
_FLAT_LEN = 692
_NB = 2


def _make_sc_call(b, l):
    info = plsc.get_sparse_core_info()
    nc, ns = info.num_cores, info.num_subcores
    nw = nc * ns
    b_per_w = b // nw
    n_chunks = b_per_w // _NB
    bc = _NB * l
    ng = bc // 16

    mesh = plsc.VectorSubcoreMesh(core_axis_name="c", subcore_axis_name="s")

    @functools.partial(
        pl.kernel,
        mesh=mesh,
        out_type=jax.ShapeDtypeStruct((b, l, _OUT_F), jnp.float32),
        compiler_params=pltpu.CompilerParams(
            needs_layout_passes=False, use_tc_tiling_on_sc=True),
        scratch_types=[
            pltpu.VMEM((_FLAT_LEN,), jnp.float32),
            pltpu.VMEM((_NB, l, _F), jnp.float32),
            pltpu.VMEM((_NB, l, _OUT_F), jnp.float32),
            pltpu.SemaphoreType.DMA,
            pltpu.SemaphoreType.DMA,
            pltpu.SemaphoreType.DMA,
            pltpu.SemaphoreType.DMA,
        ],
    )
    def sc_call(x_hbm, t_hbm, out_hbm, t_v, in_v, out_v,
                sem_ia, sem_ib, sem_oa, sem_ob):
        wid = lax.axis_index("s") * nc + lax.axis_index("c")
        bbase0 = wid * b_per_w
        pltpu.sync_copy(t_hbm, t_v)

        def make_group(lo, hi, unroll):
            @plsc.parallel_loop(lo, hi, 1, unroll=unroll)
            def group_body(g):
                r16 = lax.iota(jnp.int32, 16) + g * 16
                bv = jnp.where(r16 >= l, 1, 0)
                lv = r16 - bv * l
                bases = []
                for k in range(4):
                    col = jnp.full((16,), 16 + k, jnp.int32)
                    v = plsc.load_gather(in_v, [bv, lv, col]).astype(jnp.int32)
                    bases.append(_FLAT_OFF[k] + v * _DIMS[k])
                for c in range(_F):
                    colv = jnp.full((16,), c, jnp.int32)
                    vals = plsc.load_gather(in_v, [bv, lv, colv])
                    plsc.store_scatter(out_v, [bv, lv, colv], vals)
                c = _F
                for k in range(4):
                    for j in range(_DIMS[k]):
                        vals = plsc.load_gather(t_v, [bases[k] + j])
                        colv = jnp.full((16,), c, jnp.int32)
                        plsc.store_scatter(out_v, [bv, lv, colv], vals)
                        c += 1

        def chunk_body(ci, carry):
            bbase = bbase0 + ci * _NB
            ha = pltpu.async_copy(x_hbm.at[bbase], in_v.at[0], sem_ia)
            hb = pltpu.async_copy(x_hbm.at[bbase + 1], in_v.at[1], sem_ib)
            ha.wait()
            make_group(0, (l // 16) * 16 // 16, 2)
            hb.wait()
            make_group(12, 13, 1)
            oa = pltpu.async_copy(out_v.at[0], out_hbm.at[bbase], sem_oa)
            make_group(13, ng, 2)
            ob = pltpu.async_copy(out_v.at[1], out_hbm.at[bbase + 1], sem_ob)
            oa.wait()
            ob.wait()
            return carry

        lax.fori_loop(0, n_chunks, chunk_body, 0, unroll=False)

    return sc_call


def kernel(data, year_table, month_table, day_table, hour_table):
    b, l, f = data.shape
    t = jnp.concatenate([
        year_table.reshape(-1), month_table.reshape(-1),
        day_table.reshape(-1), hour_table.reshape(-1)])
    return _make_sc_call(b, l)(data, t)


# speedup vs baseline: 1.0471x; 1.0302x over previous
import functools

import jax
import jax.numpy as jnp
from jax import lax
from jax.experimental import pallas as pl
from jax.experimental.pallas import tpu as pltpu
from jax.experimental.pallas import tpu_sc as plsc

_F = 20
_OUT_F = 49
_DIMS = (1, 6, 12, 10)
_FLAT_OFF = (0, 2, 80, 452)
_FLAT_LEN = 692


def _make_sc_call(n, bc):
    info = plsc.get_sparse_core_info()
    nc, ns = info.num_cores, info.num_subcores
    nw = nc * ns
    n_per_w = n // nw
    n_chunks = n_per_w // bc

    mesh = plsc.VectorSubcoreMesh(core_axis_name="c", subcore_axis_name="s")

    @functools.partial(
        pl.kernel,
        mesh=mesh,
        out_type=jax.ShapeDtypeStruct((n, _OUT_F), jnp.float32),
        compiler_params=pltpu.CompilerParams(
            needs_layout_passes=False, use_tc_tiling_on_sc=True),
        scratch_types=[
            pltpu.VMEM((_FLAT_LEN,), jnp.float32),
            pltpu.VMEM((bc, _F), jnp.float32),
            pltpu.VMEM((bc, _OUT_F), jnp.float32),
            pltpu.VMEM((4, bc), jnp.int32),
        ],
    )
    def sc_call(x_hbm, t_hbm, out_hbm, t_v, in_v, out_v, base_v):
        wid = lax.axis_index("s") * nc + lax.axis_index("c")
        base0 = wid * n_per_w
        pltpu.sync_copy(t_hbm, t_v)

        def chunk_body(ci, carry):
            base = base0 + ci * bc
            pltpu.sync_copy(x_hbm.at[pl.ds(base, bc)], in_v)

            @plsc.parallel_loop(0, bc // 16, 1, unroll=8)
            def idx_body(g):
                r16 = lax.iota(jnp.int32, 16) + g * 16
                for k in range(4):
                    col = jnp.full((16,), 16 + k, jnp.int32)
                    v = plsc.load_gather(in_v, [r16, col]).astype(jnp.int32)
                    base_v[k, pl.ds(g * 16, 16)] = _FLAT_OFF[k] + v * _DIMS[k]

            @plsc.parallel_loop(0, bc // 16, 1, unroll=4)
            def emb_body(g):
                r16 = lax.iota(jnp.int32, 16) + g * 16
                for c in range(_F):
                    colv = jnp.full((16,), c, jnp.int32)
                    vals = plsc.load_gather(in_v, [r16, colv])
                    plsc.store_scatter(out_v, [r16, colv], vals)
                c = _F
                for k in range(4):
                    bk = base_v[k, pl.ds(g * 16, 16)]
                    for j in range(_DIMS[k]):
                        vals = plsc.load_gather(t_v, [bk + j])
                        colv = jnp.full((16,), c, jnp.int32)
                        plsc.store_scatter(out_v, [r16, colv], vals)
                        c += 1

            pltpu.sync_copy(out_v, out_hbm.at[pl.ds(base, bc)])
            return carry

        lax.fori_loop(0, n_chunks, chunk_body, 0, unroll=False)

    return sc_call


def kernel(data, year_table, month_table, day_table, hour_table):
    b, l, f = data.shape
    n = b * l
    t = jnp.concatenate([
        year_table.reshape(-1), month_table.reshape(-1),
        day_table.reshape(-1), hour_table.reshape(-1)])
    x2 = data.reshape(n, f)
    out = _make_sc_call(n, 256)(x2, t)
    return out.reshape(b, l, _OUT_F)
